# baseline (speedup 1.0000x reference)
import functools

import jax
import jax.numpy as jnp
from jax import lax
from jax.experimental import pallas as pl
from jax.experimental.pallas import tpu as pltpu
from jax.experimental.pallas import tpu_sc as plsc

N_CODES = 8192
DIM = 32
AUG = 40
N_ROWS = 8192
KB = 2048
RB = 1024

_NC = 2
_NS = 16
_NW = _NC * _NS
_BPW = N_ROWS // _NW
_CHUNK = 128

_BIG = 3e38


def _argmin_body(z_ref, cb0_ref, cb1_ref, cb2_ref, cb3_ref,
                 idx_ref, loss_ref, acc_ref):
    b = pl.program_id(0)
    zb = z_ref[0]
    zn = jnp.sum(zb * zb, axis=0, keepdims=True)

    m = jnp.full((8, RB), _BIG, jnp.float32)
    a = jnp.zeros((8, RB), jnp.float32)
    for q, cb_ref in enumerate((cb0_ref, cb1_ref, cb2_ref, cb3_ref)):
        cbt = cb_ref[...]
        cn = jnp.sum(cbt * cbt, axis=1, keepdims=True)
        mm2 = lax.dot_general(cbt * -2.0, zb, (((1,), (0,)), ((), ())),
                              preferred_element_type=jnp.float32)
        d = (zn + cn) + mm2
        for r in range(KB // 8):
            g = q * (KB // 8) + r
            dr = lax.slice(d, (r * 8, 0), (r * 8 + 8, RB))
            pred = dr < m
            m = jnp.minimum(m, dr)
            a = jnp.where(pred, jnp.float32(g), a)

    sio = lax.broadcasted_iota(jnp.int32, (8, RB), 0).astype(jnp.float32)
    idxv = a * 8.0 + sio
    tm = jnp.min(m, axis=0, keepdims=True)
    code = jnp.min(jnp.where(m == tm, idxv, jnp.float32(_BIG)),
                   axis=0, keepdims=True)
    idx_ref[0] = code.astype(jnp.int32)
    s = jnp.sum(tm)
    acc_ref[0] = jnp.where(b == 0, s, acc_ref[0] + s)

    @pl.when(b == pl.num_programs(0) - 1)
    def _():
        loss_ref[0, 0] = acc_ref[0]


def _argmin_call(z3, codebook, *, interpret=False):
    nb = z3.shape[0]
    cb_spec = lambda q: pl.BlockSpec((KB, DIM), lambda b: (q, 0))
    return pl.pallas_call(
        _argmin_body,
        grid=(nb,),
        in_specs=[
            pl.BlockSpec((1, DIM, RB), lambda b: (b, 0, 0)),
            cb_spec(0), cb_spec(1), cb_spec(2), cb_spec(3),
        ],
        out_specs=[
            pl.BlockSpec((1, 1, RB), lambda b: (b, 0, 0)),
            pl.BlockSpec(memory_space=pltpu.SMEM),
        ],
        out_shape=[
            jax.ShapeDtypeStruct((nb, 1, RB), jnp.int32),
            jax.ShapeDtypeStruct((1, 1), jnp.float32),
        ],
        scratch_shapes=[
            pltpu.SMEM((1,), jnp.float32),
        ],
        interpret=interpret,
    )(z3, codebook, codebook, codebook, codebook)


def _gather_body(cb_hbm, idx_hbm, zq_hbm, idx_a, idx_b, rows_v, sem):
    wid = lax.axis_index("s") * _NC + lax.axis_index("c")
    base = wid * _BPW
    pltpu.sync_copy(idx_hbm.at[wid * 2], idx_a)
    pltpu.sync_copy(idx_hbm.at[wid * 2 + 1], idx_b)
    c0 = pltpu.async_copy(cb_hbm.at[idx_a], rows_v.at[pl.ds(0, _CHUNK)], sem)
    c1 = pltpu.async_copy(cb_hbm.at[idx_b], rows_v.at[pl.ds(_CHUNK, _CHUNK)], sem)
    c0.wait()
    c1.wait()
    pltpu.sync_copy(rows_v, zq_hbm.at[pl.ds(base, _BPW)])


@functools.cache
def _gather_call():
    return pl.kernel(
        _gather_body,
        mesh=plsc.VectorSubcoreMesh(core_axis_name="c", subcore_axis_name="s"),
        out_type=jax.ShapeDtypeStruct((N_ROWS, DIM), jnp.float32),
        scratch_types=[
            pltpu.VMEM((_CHUNK,), jnp.int32),
            pltpu.VMEM((_CHUNK,), jnp.int32),
            pltpu.VMEM((_BPW, DIM), jnp.float32),
            pltpu.SemaphoreType.DMA,
        ],
        compiler_params=pltpu.CompilerParams(use_tc_tiling_on_sc=False),
    )


def kernel(z, codebook):
    b, c, h, w = z.shape
    z3 = z.reshape(b, c, h * w)
    idx8, loss_sum = _argmin_call(z3, codebook)
    idx2 = idx8.reshape(N_ROWS // _CHUNK, _CHUNK)
    zq_flat = _gather_call()(codebook, idx2)
    loss = 1.25 * loss_sum[0, 0] / jnp.float32(N_ROWS * DIM)
    zq = zq_flat.reshape(b, h, w, c).transpose(0, 3, 1, 2)
    return (zq, loss, idx8.reshape(b, 1, h, w))
